# Initial kernel scaffold; baseline (speedup 1.0000x reference)
import jax
import jax.numpy as jnp
from jax.experimental import pallas as pl


def kernel(queries, tasks, llms, Wqt, bqt, Wl, bl, Wd, bd):
    raise NotImplementedError("write your pallas kernel here")


# speedup vs baseline: 3.2108x; 3.2108x over previous
import functools

import jax
import jax.numpy as jnp
from jax import lax
from jax.experimental import pallas as pl
from jax.experimental.pallas import tpu as pltpu

IN_DIM = 1024
HID = 256
MAX_AGENT = 8
NQ = 16384
NL = 1024

RB = 256

_PREC = lax.Precision.DEFAULT


def _b16(x):
    return x.astype(jnp.bfloat16)


def _lgamma(x):
    tmp = x + 5.5
    tmp = (x + 0.5) * jnp.log(tmp) - tmp
    ser = jnp.full_like(x, 1.000000000190015)
    for j, c in enumerate((
        76.18009172947146, -86.50532032941677, 24.01409824083091,
        -1.231739572450155, 0.1208650973866179e-2, -0.5395239384953e-5,
    )):
        ser = ser + c / (x + (j + 1.0))
    return tmp + jnp.log(2.5066282746310005 * ser / x)


def _le_kernel(llms_ref, wl_ref, bl_ref, le_ref):
    le = jnp.dot(_b16(llms_ref[...]), _b16(wl_ref[...]),
                 preferred_element_type=jnp.float32, precision=_PREC)
    le = le + bl_ref[...]
    n = jnp.sqrt(jnp.sum(le * le, axis=1, keepdims=True))
    le_ref[...] = le / jnp.maximum(n, 1e-12)


def _main_kernel(q_ref, t_ref, wqt_ref, bqt_ref, wd_ref, bd_ref, le_ref,
                 rand_ref, sel_ref, logp_ref):
    x = jnp.concatenate([q_ref[...], t_ref[...]], axis=1)
    qt = jnp.dot(_b16(x), _b16(wqt_ref[...]),
                 preferred_element_type=jnp.float32, precision=_PREC)
    qt = qt + bqt_ref[...]
    n = jnp.sqrt(jnp.sum(qt * qt, axis=1, keepdims=True))
    qt = qt / jnp.maximum(n, 1e-12)

    qd_logit = jnp.sum(qt * wd_ref[...], axis=1, keepdims=True) + bd_ref[...]
    qd = jax.nn.sigmoid(qd_logit)
    lnf = qd * float(MAX_AGENT)
    lni = jnp.clip(jnp.round(lnf), 1.0, float(MAX_AGENT))

    logits = lax.dot_general(_b16(qt), _b16(le_ref[...]),
                             (((1,), (1,)), ((), ())),
                             preferred_element_type=jnp.float32,
                             precision=_PREC)
    m = jnp.max(logits, axis=1, keepdims=True)
    e = jnp.exp(logits - m)
    s = jnp.sum(e, axis=1, keepdims=True)
    scores = e / s
    CH = 128
    parts = []
    off = jnp.zeros((1, RB), jnp.float32)
    for j in range(NL // CH):
        st = jnp.transpose(scores[:, j * CH:(j + 1) * CH])
        acc = st[0:1, :]
        rows = [acc]
        for c in range(1, CH):
            acc = acc + st[c:c + 1, :]
            rows.append(acc)
        chunk_t = jnp.concatenate(rows, axis=0) + off
        parts.append(jnp.transpose(chunk_t))
        off = chunk_t[CH - 1:CH, :]
    cum = jnp.concatenate(parts, axis=1)

    cols = lax.broadcasted_iota(jnp.int32, (RB, NL), 1)
    sel = jnp.zeros((RB, NL), jnp.float32)
    sum_log_rank = jnp.zeros((RB, 1), jnp.float32)
    sum_log_s = jnp.zeros((RB, 1), jnp.float32)
    picks = []
    for i in range(1, MAX_AGENT + 1):
        r = rand_ref[:, i - 1:i]
        cnt = jnp.sum((cum <= r).astype(jnp.float32), axis=1, keepdims=True)
        idx = cnt.astype(jnp.int32)
        idx = jnp.where(idx >= NL, 0, idx)
        mask = (lni >= float(i)).astype(jnp.float32)
        onehot = cols == idx
        sel = sel + jnp.where(onehot, mask, 0.0)
        s_at = jnp.sum(jnp.where(onehot, scores, 0.0), axis=1, keepdims=True)
        sum_log_s = sum_log_s + mask * jnp.log(s_at)
        rank = mask
        for idx_p, mask_p in picks:
            rank = rank + mask_p * (idx_p == idx).astype(jnp.float32)
        sum_log_rank = sum_log_rank + mask * jnp.log(jnp.maximum(rank, 1.0))
        picks.append((idx, mask))

    sel_ref[...] = sel
    logp_ref[...] = _lgamma(lnf + 1.0) - sum_log_rank + sum_log_s


def kernel(queries, tasks, llms, Wqt, bqt, Wl, bl, Wd, bd):
    rkey = jax.random.key(42)
    rands = jnp.concatenate(
        [jax.random.uniform(jax.random.fold_in(rkey, i), (NQ, 1),
                            dtype=jnp.float32)
         for i in range(1, MAX_AGENT + 1)], axis=1)

    le = pl.pallas_call(
        _le_kernel,
        out_shape=jax.ShapeDtypeStruct((NL, HID), jnp.float32),
    )(llms, Wl, bl.reshape(1, HID))

    grid = NQ // RB
    sel, logp = pl.pallas_call(
        _main_kernel,
        grid=(grid,),
        in_specs=[
            pl.BlockSpec((RB, IN_DIM), lambda i: (i, 0)),
            pl.BlockSpec((RB, IN_DIM), lambda i: (i, 0)),
            pl.BlockSpec((2 * IN_DIM, HID), lambda i: (0, 0)),
            pl.BlockSpec((1, HID), lambda i: (0, 0)),
            pl.BlockSpec((1, HID), lambda i: (0, 0)),
            pl.BlockSpec((1, 1), lambda i: (0, 0)),
            pl.BlockSpec((NL, HID), lambda i: (0, 0)),
            pl.BlockSpec((RB, MAX_AGENT), lambda i: (i, 0)),
        ],
        out_specs=[
            pl.BlockSpec((RB, NL), lambda i: (i, 0)),
            pl.BlockSpec((RB, 1), lambda i: (i, 0)),
        ],
        out_shape=[
            jax.ShapeDtypeStruct((NQ, NL), jnp.float32),
            jax.ShapeDtypeStruct((NQ, 1), jnp.float32),
        ],
        compiler_params=pltpu.CompilerParams(
            dimension_semantics=("arbitrary",),
        ),
    )(queries, tasks, Wqt, bqt.reshape(1, HID), Wd.reshape(1, HID),
      bd.reshape(1, 1), le, rands)
    return (sel, logp)
